# Initial kernel scaffold; baseline (speedup 1.0000x reference)
import jax
import jax.numpy as jnp
from jax.experimental import pallas as pl


def kernel(self_seq, ln1_g, ln1_b, ln2_g, ln2_b, ln3_g, ln3_b, Wq, Wk, Wv, Wo, g1, wg1, wu1, wd1, g2, g2t, wg2, wu2, wd2, Wc):
    raise NotImplementedError("write your pallas kernel here")


# speedup vs baseline: 1.8441x; 1.8441x over previous
import functools

import jax
import jax.numpy as jnp
from jax.experimental import pallas as pl
from jax.experimental.pallas import tpu as pltpu

B, S, D, NH, E, K, DF = 2, 2048, 768, 12, 8, 2, 768
DO = D // 2
T = B * S
DH = D // NH
NEG = -1e9


def _ln(x, g, b):
    m = jnp.mean(x, -1, keepdims=True)
    v = jnp.mean((x - m) ** 2, -1, keepdims=True)
    return (x - m) / jnp.sqrt(v + 1e-5) * g + b


def _k1_body(x_ref, g_ref, b_ref, wq_ref, wk_ref, wv_ref, q_ref, k_ref, v_ref):
    xn = _ln(x_ref[...], g_ref[...], b_ref[...])
    q_ref[...] = jnp.dot(xn, wq_ref[...], preferred_element_type=jnp.float32)
    k_ref[...] = jnp.dot(xn, wk_ref[...], preferred_element_type=jnp.float32)
    v_ref[...] = jnp.dot(xn, wv_ref[...], preferred_element_type=jnp.float32)


def _k1(xf, g, b, Wq, Wk, Wv):
    blk = 512
    grid = (T // blk,)
    return pl.pallas_call(
        _k1_body,
        grid=grid,
        in_specs=[
            pl.BlockSpec((blk, D), lambda i: (i, 0)),
            pl.BlockSpec((1, D), lambda i: (0, 0)),
            pl.BlockSpec((1, D), lambda i: (0, 0)),
            pl.BlockSpec((D, D), lambda i: (0, 0)),
            pl.BlockSpec((D, D), lambda i: (0, 0)),
            pl.BlockSpec((D, D), lambda i: (0, 0)),
        ],
        out_specs=[
            pl.BlockSpec((blk, D), lambda i: (i, 0)),
            pl.BlockSpec((blk, D), lambda i: (i, 0)),
            pl.BlockSpec((blk, D), lambda i: (i, 0)),
        ],
        out_shape=[jax.ShapeDtypeStruct((T, D), jnp.float32)] * 3,
    )(xf, g, b, Wq, Wk, Wv)


def _k2_body(q_ref, k_ref, v_ref, o_ref, *, bq):
    qb = pl.program_id(1)
    rows = qb * bq + jax.lax.broadcasted_iota(jnp.int32, (bq, S), 0)
    cols = jax.lax.broadcasted_iota(jnp.int32, (bq, S), 1)
    causal = rows >= cols
    outs = []
    for h in range(NH):
        sl = slice(h * DH, (h + 1) * DH)
        qh = q_ref[:, sl] * jnp.float32(0.125)
        s = jax.lax.dot_general(qh, k_ref[:, sl], (((1,), (1,)), ((), ())),
                                preferred_element_type=jnp.float32)
        s = jnp.where(causal, s, NEG)
        m = jnp.max(s, -1, keepdims=True)
        p = jnp.exp(s - m)
        p = p / jnp.sum(p, -1, keepdims=True)
        outs.append(jnp.dot(p, v_ref[:, sl], preferred_element_type=jnp.float32))
    o_ref[...] = jnp.concatenate(outs, -1)


def _k2(q, k, v):
    bq = 256
    grid = (B, S // bq)
    qspec = pl.BlockSpec((bq, D), lambda b, i: (b * (S // bq) + i, 0))
    kvspec = pl.BlockSpec((S, D), lambda b, i: (b, 0))
    return pl.pallas_call(
        functools.partial(_k2_body, bq=bq),
        grid=grid,
        in_specs=[qspec, kvspec, kvspec],
        out_specs=pl.BlockSpec((bq, D), lambda b, i: (b * (S // bq) + i, 0)),
        out_shape=jax.ShapeDtypeStruct((T, D), jnp.float32),
    )(q, k, v)


def _k3_body(o_ref, wo_ref, x0_ref, g2_ref, b2_ref, g3_ref, b3_ref,
             gw1_ref, gw2_ref, x_ref, xn3_ref, l1_ref, l2_ref):
    proj = jnp.dot(o_ref[...], wo_ref[...], preferred_element_type=jnp.float32)
    x = x0_ref[...] + _ln(proj, g2_ref[...], b2_ref[...])
    x_ref[...] = x
    xn3 = _ln(x, g3_ref[...], b3_ref[...])
    xn3_ref[...] = xn3
    l1_ref[...] = jnp.dot(xn3, gw1_ref[...], preferred_element_type=jnp.float32)
    l2_ref[...] = jnp.dot(xn3, gw2_ref[...], preferred_element_type=jnp.float32)


def _k3(o, Wo, x0, g2, b2, g3, b3, gw1, gw2):
    blk = 512
    grid = (T // blk,)
    row = pl.BlockSpec((blk, D), lambda i: (i, 0))
    par = pl.BlockSpec((1, D), lambda i: (0, 0))
    return pl.pallas_call(
        _k3_body,
        grid=grid,
        in_specs=[row, pl.BlockSpec((D, D), lambda i: (0, 0)), row,
                  par, par, par, par,
                  pl.BlockSpec((D, E), lambda i: (0, 0)),
                  pl.BlockSpec((D, E), lambda i: (0, 0))],
        out_specs=[row, row,
                   pl.BlockSpec((blk, E), lambda i: (i, 0)),
                   pl.BlockSpec((blk, E), lambda i: (i, 0))],
        out_shape=[jax.ShapeDtypeStruct((T, D), jnp.float32),
                   jax.ShapeDtypeStruct((T, D), jnp.float32),
                   jax.ShapeDtypeStruct((T, E), jnp.float32),
                   jax.ShapeDtypeStruct((T, E), jnp.float32)],
    )(o, Wo, x0, g2, b2, g3, b3, gw1, gw2)


def _top2(p):
    iota = jax.lax.broadcasted_iota(jnp.int32, p.shape, 1)
    m0 = jnp.max(p, -1, keepdims=True)
    i0 = jnp.min(jnp.where(p == m0, iota, E), -1, keepdims=True)
    oh0 = (iota == i0)
    pm = jnp.where(oh0, NEG, p)
    m1 = jnp.max(pm, -1, keepdims=True)
    i1 = jnp.min(jnp.where(pm == m1, iota, E), -1, keepdims=True)
    oh1 = (iota == i1)
    return m0, oh0, m1, oh1


def _k4_body(l1_ref, l2_ref, c1_ref, c2_ref, bal_ref):
    def combine(l):
        mx = jnp.max(l, -1, keepdims=True)
        eexp = jnp.exp(l - mx)
        p = eexp / jnp.sum(eexp, -1, keepdims=True)
        m0, oh0, m1, oh1 = _top2(p)
        wsum = m0 + m1
        comb = (oh0 * m0 + oh1 * m1) / wsum
        return p, oh0, comb

    p1, oh0_1, c1 = combine(l1_ref[...])
    _, _, c2 = combine(l2_ref[...])
    c1_ref[...] = c1
    c2_ref[...] = c2
    f = jnp.mean(oh0_1.astype(jnp.float32), axis=0, keepdims=True)
    P = jnp.mean(p1, axis=0, keepdims=True)
    bal_ref[...] = jnp.float32(E) * jnp.sum(f * P, keepdims=True)


def _k4(l1, l2):
    return pl.pallas_call(
        _k4_body,
        in_specs=[pl.BlockSpec((T, E), lambda: (0, 0))] * 2,
        out_specs=[pl.BlockSpec((T, E), lambda: (0, 0)),
                   pl.BlockSpec((T, E), lambda: (0, 0)),
                   pl.BlockSpec((1, 1), lambda: (0, 0))],
        out_shape=[jax.ShapeDtypeStruct((T, E), jnp.float32),
                   jax.ShapeDtypeStruct((T, E), jnp.float32),
                   jax.ShapeDtypeStruct((1, 1), jnp.float32)],
    )(l1, l2)


def _k5_body(x_ref, wg_ref, wu_ref, wd_ref, c_ref, o_ref):
    e = pl.program_id(1)
    x = x_ref[...]
    t1 = jax.nn.leaky_relu(jnp.dot(x, wg_ref[0], preferred_element_type=jnp.float32))
    t2 = jnp.dot(x, wu_ref[0], preferred_element_type=jnp.float32)
    y = jnp.dot(t1 * t2, wd_ref[0], preferred_element_type=jnp.float32)
    lane = jax.lax.broadcasted_iota(jnp.int32, (c_ref.shape[0], E), 1)
    ce = jnp.sum(jnp.where(lane == e, c_ref[...], 0.0), -1, keepdims=True)
    contrib = ce * y

    @pl.when(e == 0)
    def _():
        o_ref[...] = contrib

    @pl.when(e > 0)
    def _():
        o_ref[...] += contrib


def _k5(xn3, wg, wu, wd, comb):
    blk = 1024
    grid = (T // blk, E)
    return pl.pallas_call(
        _k5_body,
        grid=grid,
        in_specs=[
            pl.BlockSpec((blk, D), lambda i, e: (i, 0)),
            pl.BlockSpec((1, D, DF), lambda i, e: (e, 0, 0)),
            pl.BlockSpec((1, D, DF), lambda i, e: (e, 0, 0)),
            pl.BlockSpec((1, DF, DO), lambda i, e: (e, 0, 0)),
            pl.BlockSpec((blk, E), lambda i, e: (i, 0)),
        ],
        out_specs=pl.BlockSpec((blk, DO), lambda i, e: (i, 0)),
        out_shape=jax.ShapeDtypeStruct((T, DO), jnp.float32),
        compiler_params=pltpu.CompilerParams(
            dimension_semantics=("parallel", "arbitrary")),
    )(xn3, wg, wu, wd, comb)


def _k6_body(x_ref, o1_ref, o2_ref, out_ref):
    out_ref[...] = x_ref[...] + jnp.concatenate([o1_ref[...], o2_ref[...]], -1)


def _k6(x, o1, o2):
    blk = 1024
    grid = (T // blk,)
    return pl.pallas_call(
        _k6_body,
        grid=grid,
        in_specs=[pl.BlockSpec((blk, D), lambda i: (i, 0)),
                  pl.BlockSpec((blk, DO), lambda i: (i, 0)),
                  pl.BlockSpec((blk, DO), lambda i: (i, 0))],
        out_specs=pl.BlockSpec((blk, D), lambda i: (i, 0)),
        out_shape=jax.ShapeDtypeStruct((T, D), jnp.float32),
    )(x, o1, o2)


def kernel(self_seq, ln1_g, ln1_b, ln2_g, ln2_b, ln3_g, ln3_b,
           Wq, Wk, Wv, Wo, g1, wg1, wu1, wd1, g2, g2t, wg2, wu2, wd2, Wc):
    xf = self_seq.reshape(T, D)
    q, k, v = _k1(xf, ln1_g.reshape(1, D), ln1_b.reshape(1, D), Wq, Wk, Wv)
    o = _k2(q, k, v)
    x, xn3, l1, l2 = _k3(o, Wo, xf, ln2_g.reshape(1, D), ln2_b.reshape(1, D),
                         ln3_g.reshape(1, D), ln3_b.reshape(1, D), g1, g2)
    c1, c2, bal = _k4(l1, l2)
    o1 = _k5(xn3, wg1, wu1, wd1, c1)
    o2 = _k5(xn3, wg2, wu2, wd2, c2)
    out = _k6(x, o1, o2)
    return (out.reshape(B, S, D), bal.reshape(()), jnp.zeros((), jnp.float32))
